# Initial kernel scaffold; baseline (speedup 1.0000x reference)
import jax
import jax.numpy as jnp
from jax.experimental import pallas as pl


def kernel(sparse_inputs, tables, fixed_vector):
    raise NotImplementedError("write your pallas kernel here")


# speedup vs baseline: 1.0922x; 1.0922x over previous
import functools

import jax
import jax.numpy as jnp
from jax import lax
from jax.experimental import pallas as pl
from jax.experimental.pallas import tpu as pltpu
from jax.experimental.pallas import tpu_sc as plsc

_NUM_FIELDS = 26
_VOCAB = 100000
_DIM = 32
_BATCH = 16384

_NC = 2
_NS = 16
_L = 16

_TOTAL = _BATCH * _NUM_FIELDS
_PER_SUB = _TOTAL // _NS
_CHUNK = 128
_ROWS_PER_SUB = _PER_SUB // _CHUNK
_PER_TILE = _PER_SUB // _NC
_CHUNKS_PER_TILE = _PER_TILE // _CHUNK
_VECS_PER_ROW = _CHUNK // _L


def _body(idx_hbm, table_hbm, fixed_hbm, out_hbm,
          idx_v, acc_v, buf0, buf1, fixed_v, masked_v, sums_v, shared,
          gs0, gs1, ws0, ws1):
    c = lax.axis_index("c")
    s = lax.axis_index("s")
    base = s * _PER_SUB

    pltpu.sync_copy(idx_hbm.at[pl.ds(s * _ROWS_PER_SUB, _ROWS_PER_SUB)], idx_v)
    pltpu.sync_copy(fixed_hbm, fixed_v)

    zero16 = jnp.zeros((_L,), jnp.int32)
    acc_v[pl.ds(0, _L)] = zero16
    acc_v[pl.ds(_L, _L)] = zero16
    lanes = lax.iota(jnp.int32, _L)

    def pass1(r, carry):
        p0 = base + r * _CHUNK
        for q in range(_VECS_PER_ROW):
            off = q * _L
            v = idx_v[r, pl.ds(off, _L)]
            field = lax.rem(p0 + off + lanes, _NUM_FIELDS)
            plsc.addupdate_scatter(acc_v, [field], v)
            idx_v[r, pl.ds(off, _L)] = v + field * _VOCAB
        return carry

    lax.fori_loop(0, _ROWS_PER_SUB, pass1, 0)

    pltpu.sync_copy(acc_v, shared.at[s])
    plsc.subcore_barrier()
    pltpu.sync_copy(shared, sums_v)
    t0 = jnp.zeros((_L,), jnp.int32)
    t1 = jnp.zeros((_L,), jnp.int32)
    for r in range(_NS):
        t0 = t0 + sums_v[r, pl.ds(0, _L)]
        t1 = t1 + sums_v[r, pl.ds(_L, _L)]
    m0 = jnp.where((t0 == 0) & (lanes < _NUM_FIELDS), 1, 0).astype(jnp.int32)
    m1 = jnp.where((t1 == 0) & (lanes + _L < _NUM_FIELDS), 1, 0).astype(jnp.int32)
    masked_v[pl.ds(0, _L)] = m0
    masked_v[pl.ds(_L, _L)] = m1
    any_masked = (jnp.sum(m0) + jnp.sum(m1)) > 0

    idx_row0 = c * _CHUNKS_PER_TILE
    out_base = base + c * _PER_TILE

    def fix(buf, row_start):
        def fix_row(r, carry):
            f = lax.rem(row_start + r, _NUM_FIELDS)
            flag = masked_v[pl.ds(f, _L)][0]

            @pl.when(flag != 0)
            def _():
                buf[r, pl.ds(0, _L)] = fixed_v[pl.ds(0, _L)]
                buf[r, pl.ds(_L, _L)] = fixed_v[pl.ds(_L, _L)]

            return carry

        lax.fori_loop(0, _CHUNK, fix_row, 0)

    def gather_pair(g, carry):
        k0 = 2 * g
        r0 = out_base + k0 * _CHUNK
        r1 = r0 + _CHUNK

        @pl.when(g > 0)
        def _():
            pltpu.make_async_copy(buf0, out_hbm.at[pl.ds(r0, _CHUNK)], ws0).wait()

        pltpu.make_async_copy(
            table_hbm.at[idx_v.at[idx_row0 + k0]], buf0, gs0).start()

        @pl.when(g > 0)
        def _():
            pltpu.make_async_copy(buf1, out_hbm.at[pl.ds(r1, _CHUNK)], ws1).wait()

        pltpu.make_async_copy(
            table_hbm.at[idx_v.at[idx_row0 + k0 + 1]], buf1, gs1).start()

        pltpu.make_async_copy(
            table_hbm.at[idx_v.at[idx_row0 + k0]], buf0, gs0).wait()

        @pl.when(any_masked)
        def _():
            fix(buf0, r0)

        pltpu.make_async_copy(buf0, out_hbm.at[pl.ds(r0, _CHUNK)], ws0).start()

        pltpu.make_async_copy(
            table_hbm.at[idx_v.at[idx_row0 + k0 + 1]], buf1, gs1).wait()

        @pl.when(any_masked)
        def _():
            fix(buf1, r1)

        pltpu.make_async_copy(buf1, out_hbm.at[pl.ds(r1, _CHUNK)], ws1).start()
        return carry

    lax.fori_loop(0, _CHUNKS_PER_TILE // 2, gather_pair, 0)
    pltpu.make_async_copy(buf0, out_hbm.at[pl.ds(out_base, _CHUNK)], ws0).wait()
    pltpu.make_async_copy(buf1, out_hbm.at[pl.ds(out_base, _CHUNK)], ws1).wait()


@functools.partial(
    pl.kernel,
    out_type=jax.ShapeDtypeStruct((_TOTAL, _DIM), jnp.float32),
    mesh=plsc.VectorSubcoreMesh(core_axis_name="c", subcore_axis_name="s"),
    compiler_params=pltpu.CompilerParams(
        needs_layout_passes=False, use_tc_tiling_on_sc=False),
    scratch_types=[
        pltpu.VMEM((_ROWS_PER_SUB, _CHUNK), jnp.int32),
        pltpu.VMEM((2 * _L,), jnp.int32),
        pltpu.VMEM((_CHUNK, _DIM), jnp.float32),
        pltpu.VMEM((_CHUNK, _DIM), jnp.float32),
        pltpu.VMEM((_DIM,), jnp.float32),
        pltpu.VMEM((3 * _L,), jnp.int32),
        pltpu.VMEM((_NS, 2 * _L), jnp.int32),
        pltpu.VMEM_SHARED((_NS, 2 * _L), jnp.int32),
        pltpu.SemaphoreType.DMA,
        pltpu.SemaphoreType.DMA,
        pltpu.SemaphoreType.DMA,
        pltpu.SemaphoreType.DMA,
    ],
)
def _sc_embedding(idx_hbm, table_hbm, fixed_hbm, out_hbm,
                  idx_v, acc_v, buf0, buf1, fixed_v, masked_v, sums_v, shared,
                  gs0, gs1, ws0, ws1):
    _body(idx_hbm, table_hbm, fixed_hbm, out_hbm,
          idx_v, acc_v, buf0, buf1, fixed_v, masked_v, sums_v, shared,
          gs0, gs1, ws0, ws1)


def kernel(sparse_inputs, tables, fixed_vector):
    idx = sparse_inputs.astype(jnp.int32).reshape(_TOTAL // _CHUNK, _CHUNK)
    table_flat = tables.reshape(_NUM_FIELDS * _VOCAB, _DIM)
    fixed = fixed_vector.astype(jnp.float32)
    out = _sc_embedding(idx, table_flat, fixed)
    return out.reshape(_BATCH, _NUM_FIELDS, _DIM)
